# baseline (device time: 182852 ns/iter reference)
import jax
import jax.numpy as jnp
from jax import lax
from jax.experimental import pallas as pl
from jax.experimental.pallas import tpu as pltpu

N_DEV = 8


def _gelu(y):
    c = 0.7978845608028654
    return 0.5 * y * (1.0 + jnp.tanh(c * (y + 0.044715 * y * y * y)))


def kernel(x, w_mat):
    m_total, k_shard = x.shape
    _, n = w_mat.shape
    m_per = m_total // N_DEV

    def body(x_ref, w_ref, out_ref, partial_ref, comm_ref, send_sems, recv_sems):
        my = lax.axis_index("i")
        left = lax.rem(my + N_DEV - 1, N_DEV)
        right = lax.rem(my + 1, N_DEV)

        barrier_sem = pltpu.get_barrier_semaphore()
        for nbr in (left, right):
            pl.semaphore_signal(
                barrier_sem, inc=1,
                device_id=(nbr,), device_id_type=pl.DeviceIdType.MESH,
            )
        pl.semaphore_wait(barrier_sem, 2)

        partial_ref[...] = jnp.dot(
            x_ref[...], w_ref[...], preferred_element_type=jnp.float32
        )

        c0 = lax.rem(my + N_DEV - 1, N_DEV)
        comm_ref[0] = partial_ref[pl.ds(c0 * m_per, m_per), :]

        for h in range(N_DEV - 1):
            rdma = pltpu.make_async_remote_copy(
                src_ref=comm_ref.at[h],
                dst_ref=comm_ref.at[h + 1],
                send_sem=send_sems.at[h],
                recv_sem=recv_sems.at[h],
                device_id=(right,),
                device_id_type=pl.DeviceIdType.MESH,
            )
            rdma.start()
            rdma.wait()
            c = lax.rem(my + 2 * N_DEV - h - 2, N_DEV)
            comm_ref[h + 1] = comm_ref[h + 1] + partial_ref[pl.ds(c * m_per, m_per), :]

        out_ref[...] = _gelu(comm_ref[N_DEV - 1])

    return pl.pallas_call(
        body,
        out_shape=jax.ShapeDtypeStruct((m_per, n), jnp.float32),
        in_specs=[
            pl.BlockSpec(memory_space=pltpu.VMEM),
            pl.BlockSpec(memory_space=pltpu.VMEM),
        ],
        out_specs=pl.BlockSpec(memory_space=pltpu.VMEM),
        scratch_shapes=[
            pltpu.VMEM((m_total, n), jnp.float32),
            pltpu.VMEM((N_DEV, m_per, n), jnp.float32),
            pltpu.SemaphoreType.DMA((N_DEV - 1,)),
            pltpu.SemaphoreType.DMA((N_DEV - 1,)),
        ],
        compiler_params=pltpu.CompilerParams(collective_id=0),
    )(x, w_mat)


# device time: 104952 ns/iter; 1.7422x vs baseline; 1.7422x over previous
import jax
import jax.numpy as jnp
from jax import lax
from jax.experimental import pallas as pl
from jax.experimental.pallas import tpu as pltpu

N_DEV = 8


def _gelu(y):
    c = 0.7978845608028654
    return 0.5 * y * (1.0 + jnp.tanh(c * (y + 0.044715 * y * y * y)))


def kernel(x, w_mat):
    m_total, k_shard = x.shape
    _, n = w_mat.shape
    m_per = m_total // N_DEV
    nh = n // 2

    def body(x_ref, w_ref, out_ref, commR, commL, pR, pL,
             send_semsR, recv_semsR, send_semsL, recv_semsL):
        my = lax.axis_index("i")
        left = lax.rem(my + N_DEV - 1, N_DEV)
        right = lax.rem(my + 1, N_DEV)

        barrier_sem = pltpu.get_barrier_semaphore()
        for nbr in (left, right):
            pl.semaphore_signal(
                barrier_sem, inc=1,
                device_id=(nbr,), device_id_type=pl.DeviceIdType.MESH,
            )
        pl.semaphore_wait(barrier_sem, 2)

        def dotR(c):
            return jnp.dot(x_ref[pl.ds(c * m_per, m_per), :], w_ref[:, :nh],
                           preferred_element_type=jnp.float32)

        def dotL(c):
            return jnp.dot(x_ref[pl.ds(c * m_per, m_per), :], w_ref[:, nh:],
                           preferred_element_type=jnp.float32)

        commR[0] = dotR(lax.rem(my + N_DEV - 1, N_DEV))
        commL[0] = dotL(lax.rem(my + 1, N_DEV))

        for h in range(N_DEV - 1):
            rdmaR = pltpu.make_async_remote_copy(
                src_ref=commR.at[h],
                dst_ref=commR.at[h + 1],
                send_sem=send_semsR.at[h],
                recv_sem=recv_semsR.at[h],
                device_id=(right,),
                device_id_type=pl.DeviceIdType.MESH,
            )
            rdmaL = pltpu.make_async_remote_copy(
                src_ref=commL.at[h],
                dst_ref=commL.at[h + 1],
                send_sem=send_semsL.at[h],
                recv_sem=recv_semsL.at[h],
                device_id=(left,),
                device_id_type=pl.DeviceIdType.MESH,
            )
            rdmaR.start()
            rdmaL.start()
            pR[...] = dotR(lax.rem(my + 2 * N_DEV - h - 2, N_DEV))
            pL[...] = dotL(lax.rem(my + h + 2, N_DEV))
            rdmaR.wait()
            rdmaL.wait()
            commR[h + 1] = commR[h + 1] + pR[...]
            commL[h + 1] = commL[h + 1] + pL[...]

        out_ref[:, :nh] = _gelu(commR[N_DEV - 1])
        out_ref[:, nh:] = _gelu(commL[N_DEV - 1])

    return pl.pallas_call(
        body,
        out_shape=jax.ShapeDtypeStruct((m_per, n), jnp.float32),
        in_specs=[
            pl.BlockSpec(memory_space=pltpu.VMEM),
            pl.BlockSpec(memory_space=pltpu.VMEM),
        ],
        out_specs=pl.BlockSpec(memory_space=pltpu.VMEM),
        scratch_shapes=[
            pltpu.VMEM((N_DEV, m_per, nh), jnp.float32),
            pltpu.VMEM((N_DEV, m_per, nh), jnp.float32),
            pltpu.VMEM((m_per, nh), jnp.float32),
            pltpu.VMEM((m_per, nh), jnp.float32),
            pltpu.SemaphoreType.DMA((N_DEV - 1,)),
            pltpu.SemaphoreType.DMA((N_DEV - 1,)),
            pltpu.SemaphoreType.DMA((N_DEV - 1,)),
            pltpu.SemaphoreType.DMA((N_DEV - 1,)),
        ],
        compiler_params=pltpu.CompilerParams(collective_id=0),
    )(x, w_mat)


# device time: 90985 ns/iter; 2.0097x vs baseline; 1.1535x over previous
import jax
import jax.numpy as jnp
from jax import lax
from jax.experimental import pallas as pl
from jax.experimental.pallas import tpu as pltpu

N_DEV = 8
N_SUB = 2


def _gelu(y):
    c = 0.7978845608028654
    return 0.5 * y * (1.0 + jnp.tanh(c * (y + 0.044715 * y * y * y)))


def kernel(x, w_mat):
    m_total, k_shard = x.shape
    _, n = w_mat.shape
    m_per = m_total // N_DEV
    nh = n // 2
    nq = nh // N_SUB

    def body(x_ref, w_ref, out_ref, commR, commL, pR, pL,
             send_semsR, recv_semsR, send_semsL, recv_semsL):
        my = lax.axis_index("i")
        left = lax.rem(my + N_DEV - 1, N_DEV)
        right = lax.rem(my + 1, N_DEV)

        barrier_sem = pltpu.get_barrier_semaphore()
        for nbr in (left, right):
            pl.semaphore_signal(
                barrier_sem, inc=1,
                device_id=(nbr,), device_id_type=pl.DeviceIdType.MESH,
            )
        pl.semaphore_wait(barrier_sem, 2)

        def x_rows(c):
            return x_ref[pl.ds(c * m_per, m_per), :]

        def chunkR(h):
            return lax.rem(my + 2 * N_DEV - h - 1, N_DEV)

        def chunkL(h):
            return lax.rem(my + h + 1, N_DEV)

        def make_rdma(comm, sends, recvs, h, b, dst):
            return pltpu.make_async_remote_copy(
                src_ref=comm.at[h, b],
                dst_ref=comm.at[h + 1, b],
                send_sem=sends.at[h, b],
                recv_sem=recvs.at[h, b],
                device_id=(dst,),
                device_id_type=pl.DeviceIdType.MESH,
            )

        xr0 = x_rows(chunkR(0))
        xl0 = x_rows(chunkL(0))
        for b in range(N_SUB):
            commR[0, b] = jnp.dot(xr0, w_ref[:, b * nq:(b + 1) * nq],
                                  preferred_element_type=jnp.float32)
            commL[0, b] = jnp.dot(xl0, w_ref[:, nh + b * nq:nh + (b + 1) * nq],
                                  preferred_element_type=jnp.float32)
        for b in range(N_SUB):
            make_rdma(commR, send_semsR, recv_semsR, 0, b, right).start()
            make_rdma(commL, send_semsL, recv_semsL, 0, b, left).start()

        for h in range(N_DEV - 1):
            xr = x_rows(chunkR(h + 1))
            xl = x_rows(chunkL(h + 1))
            for b in range(N_SUB):
                pR[b] = jnp.dot(xr, w_ref[:, b * nq:(b + 1) * nq],
                                preferred_element_type=jnp.float32)
                pL[b] = jnp.dot(xl, w_ref[:, nh + b * nq:nh + (b + 1) * nq],
                                preferred_element_type=jnp.float32)
            for b in range(N_SUB):
                make_rdma(commR, send_semsR, recv_semsR, h, b, right).wait()
                commR[h + 1, b] = commR[h + 1, b] + pR[b]
                if h < N_DEV - 2:
                    make_rdma(commR, send_semsR, recv_semsR, h + 1, b,
                              right).start()
                make_rdma(commL, send_semsL, recv_semsL, h, b, left).wait()
                commL[h + 1, b] = commL[h + 1, b] + pL[b]
                if h < N_DEV - 2:
                    make_rdma(commL, send_semsL, recv_semsL, h + 1, b,
                              left).start()

        for b in range(N_SUB):
            out_ref[:, b * nq:(b + 1) * nq] = _gelu(commR[N_DEV - 1, b])
            out_ref[:, nh + b * nq:nh + (b + 1) * nq] = _gelu(
                commL[N_DEV - 1, b])

    return pl.pallas_call(
        body,
        out_shape=jax.ShapeDtypeStruct((m_per, n), jnp.float32),
        in_specs=[
            pl.BlockSpec(memory_space=pltpu.VMEM),
            pl.BlockSpec(memory_space=pltpu.VMEM),
        ],
        out_specs=pl.BlockSpec(memory_space=pltpu.VMEM),
        scratch_shapes=[
            pltpu.VMEM((N_DEV, N_SUB, m_per, nq), jnp.float32),
            pltpu.VMEM((N_DEV, N_SUB, m_per, nq), jnp.float32),
            pltpu.VMEM((N_SUB, m_per, nq), jnp.float32),
            pltpu.VMEM((N_SUB, m_per, nq), jnp.float32),
            pltpu.SemaphoreType.DMA((N_DEV - 1, N_SUB)),
            pltpu.SemaphoreType.DMA((N_DEV - 1, N_SUB)),
            pltpu.SemaphoreType.DMA((N_DEV - 1, N_SUB)),
            pltpu.SemaphoreType.DMA((N_DEV - 1, N_SUB)),
        ],
        compiler_params=pltpu.CompilerParams(collective_id=0),
    )(x, w_mat)
